# baseline (speedup 1.0000x reference)
import functools

import jax
import jax.numpy as jnp
from jax import lax
from jax.experimental import pallas as pl
from jax.experimental.pallas import tpu as pltpu
from jax.experimental.pallas import tpu_sc as plsc

N_NODES = 10000
N_EDGES = 20000
NNZ = 320000
HID = 128
EW = 16
NQ = HID // EW

NS = 16
PER_W = NNZ // NS
CHUNK = 1000
ITERS = PER_W // CHUNK
OUT_SLICE = 2000

_mesh = plsc.VectorSubcoreMesh(core_axis_name="c", subcore_axis_name="s")

_SC_PARAMS = pltpu.CompilerParams(use_tc_tiling_on_sc=False)


def _f32(shape):
    return jax.ShapeDtypeStruct(shape, jnp.float32)


def _zero_acc(acc, zeros_hbm, s):
    @pl.when(s < N_EDGES // OUT_SLICE)
    def _():
        pltpu.sync_copy(zeros_hbm, acc.at[pl.ds(s * OUT_SLICE, OUT_SLICE)])


def _copy_out(acc, out, s, cond):
    @pl.when(cond)
    def _():
        sl = pl.ds(s * OUT_SLICE, OUT_SLICE)
        pltpu.sync_copy(acc.at[sl], out.at[sl])


def _load_idx2d(idx_hbm, buf, s):
    base = s * PER_W

    @pl.loop(0, ITERS)
    def _(i):
        pltpu.sync_copy(idx_hbm.at[pl.ds(base + i * CHUNK, CHUNK)], buf.at[i])


def _run_phase(tbl, acc, src2d, dst2d, rows0, rows1, sem0, sem1):
    def gather(i, buf, sem):
        pltpu.async_copy(tbl.at[src2d.at[i]], buf, sem)

    def wait_scatter(i, buf, sem):
        pltpu.make_async_copy(tbl.at[src2d.at[i]], buf, sem).wait()
        pltpu.sync_copy(buf, acc.at[dst2d.at[i]], add=True)

    gather(0, rows0, sem0)

    @pl.loop(0, ITERS, step=2)
    def _(i):
        gather(i + 1, rows1, sem1)
        wait_scatter(i, rows0, sem0)

        @pl.when(i + 2 < ITERS)
        def _():
            gather(i + 2, rows0, sem0)

        wait_scatter(i + 1, rows1, sem1)


_SC_SCRATCH = [
    pltpu.VMEM((ITERS, CHUNK), jnp.int32),
    pltpu.VMEM((ITERS, CHUNK), jnp.int32),
    pltpu.VMEM((CHUNK, EW), jnp.float32),
    pltpu.VMEM((CHUNK, EW), jnp.float32),
    pltpu.VMEM_SHARED((N_EDGES, EW), jnp.float32),
    pltpu.SemaphoreType.DMA,
    pltpu.SemaphoreType.DMA,
]


@functools.partial(
    pl.kernel,
    out_type=(_f32((N_EDGES, EW)), _f32((N_NODES, EW))),
    mesh=_mesh,
    compiler_params=_SC_PARAMS,
    scratch_types=_SC_SCRATCH,
)
def _k_hist(tbl_n, tbl_e, nidx, eidx, zeros_hbm, oute, outn,
            buf_n, buf_e, rows0, rows1, acc, sem0, sem1):
    c = lax.axis_index("c")
    s = lax.axis_index("s")

    _zero_acc(acc, zeros_hbm, s)
    _load_idx2d(nidx, buf_n, s)
    _load_idx2d(eidx, buf_e, s)
    plsc.subcore_barrier()

    pl.when(c == 0)(lambda: _run_phase(
        tbl_n, acc, buf_n, buf_e, rows0, rows1, sem0, sem1))
    pl.when(c == 1)(lambda: _run_phase(
        tbl_e, acc, buf_e, buf_n, rows0, rows1, sem0, sem1))
    plsc.subcore_barrier()

    _copy_out(acc, oute, s, jnp.logical_and(c == 0, s < N_EDGES // OUT_SLICE))
    _copy_out(acc, outn, s, jnp.logical_and(c == 1, s < N_NODES // OUT_SLICE))


def _make_pass(with_norm):
    outs = tuple(_f32((N_EDGES, EW)) for _ in range(NQ))
    if with_norm:
        outs = outs + (_f32((N_EDGES, EW)), _f32((N_NODES, EW)))

    @functools.partial(
        pl.kernel,
        out_type=outs,
        mesh=_mesh,
        compiler_params=_SC_PARAMS,
        scratch_types=_SC_SCRATCH,
    )
    def k(*refs):
        t = refs[:NQ]
        if with_norm:
            tbl_n, tbl_e = refs[NQ], refs[NQ + 1]
            sidx, didx, zeros_hbm = refs[NQ + 2:NQ + 5]
            o = refs[NQ + 5:2 * NQ + 5]
            od1, od0 = refs[2 * NQ + 5], refs[2 * NQ + 6]
            buf_s, buf_d, rows0, rows1, acc, sem0, sem1 = refs[2 * NQ + 7:]
        else:
            sidx, didx, zeros_hbm = refs[NQ:NQ + 3]
            o = refs[NQ + 3:2 * NQ + 3]
            buf_s, buf_d, rows0, rows1, acc, sem0, sem1 = refs[2 * NQ + 3:]

        c = lax.axis_index("c")
        s = lax.axis_index("s")
        is0 = jnp.logical_and(c == 0, s < N_EDGES // OUT_SLICE)
        is1 = jnp.logical_and(c == 1, s < N_EDGES // OUT_SLICE)

        _zero_acc(acc, zeros_hbm, s)
        _load_idx2d(sidx, buf_s, s)
        _load_idx2d(didx, buf_d, s)
        plsc.subcore_barrier()

        if with_norm:
            pl.when(c == 0)(lambda: _run_phase(
                tbl_n, acc, buf_s, buf_d, rows0, rows1, sem0, sem1))
            pl.when(c == 1)(lambda: _run_phase(
                tbl_e, acc, buf_d, buf_s, rows0, rows1, sem0, sem1))
            plsc.subcore_barrier()
            _copy_out(acc, od1, s, is0)
            _copy_out(acc, od0, s,
                      jnp.logical_and(c == 1, s < N_NODES // OUT_SLICE))
            plsc.subcore_barrier()
            _zero_acc(acc, zeros_hbm, s)
            plsc.subcore_barrier()

        for p in range(NQ // 2):
            pl.when(c == 0)(functools.partial(
                _run_phase, t[2 * p], acc, buf_s, buf_d,
                rows0, rows1, sem0, sem1))
            pl.when(c == 1)(functools.partial(
                _run_phase, t[2 * p + 1], acc, buf_s, buf_d,
                rows0, rows1, sem0, sem1))
            plsc.subcore_barrier()
            _copy_out(acc, o[2 * p], s, is0)
            _copy_out(acc, o[2 * p + 1], s, is1)
            if p != NQ // 2 - 1:
                plsc.subcore_barrier()
                _zero_acc(acc, zeros_hbm, s)
                plsc.subcore_barrier()

    return k


_k_pass = _make_pass(False)
_k_pass_norm = _make_pass(True)


def _norm_body(ec_ref, nc_ref, te_ref, tn_ref):
    r = lax.rsqrt(jnp.maximum(ec_ref[...], 1.0))
    te_ref[...] = r * r * r
    tn_ref[...] = lax.rsqrt(jnp.maximum(nc_ref[...], 1.0))


_k_norm = pl.pallas_call(
    _norm_body,
    out_shape=(_f32((N_EDGES, EW)), _f32((N_NODES, EW))),
)


_BR = 1000


def _write_slices(y, out_refs):
    for j, o_ref in enumerate(out_refs):
        o_ref[...] = y[:, j * EW:(j + 1) * EW]


def _in_body(x_ref, w_ref, t_ref, *out_refs):
    y = jnp.dot(x_ref[...], w_ref[...], preferred_element_type=jnp.float32,
                precision=lax.Precision.HIGHEST)
    _write_slices(y * t_ref[:, 0:1], out_refs)


_q_specs = tuple(pl.BlockSpec((_BR, EW), lambda i: (i, 0)) for _ in range(NQ))
_q_shapes = tuple(_f32((N_EDGES, EW)) for _ in range(NQ))

_k_in = pl.pallas_call(
    _in_body,
    grid=(N_NODES // _BR,),
    in_specs=[
        pl.BlockSpec((_BR, HID), lambda i: (i, 0)),
        pl.BlockSpec((HID, HID), lambda i: (0, 0)),
        pl.BlockSpec((_BR, EW), lambda i: (i, 0)),
    ],
    out_specs=_q_specs,
    out_shape=_q_shapes,
)


def _mid_body(*refs):
    a = refs[:NQ]
    d_ref, b_ref, w_ref, t_ref = refs[NQ:NQ + 4]
    out_refs = refs[NQ + 4:]
    dinv = 1.0 / jnp.maximum(d_ref[:, 0:1], 1e-12)
    x = jnp.concatenate([r[...] for r in a], axis=1)
    x1 = jax.nn.sigmoid(x * dinv + b_ref[...])
    y = jnp.dot(x1, w_ref[...], preferred_element_type=jnp.float32,
                precision=lax.Precision.HIGHEST)
    _write_slices(y * t_ref[:, 0:1], out_refs)


def _make_mid(n_rows):
    return pl.pallas_call(
        _mid_body,
        grid=(n_rows // _BR,),
        in_specs=[
            *(pl.BlockSpec((_BR, EW), lambda i: (i, 0)) for _ in range(NQ)),
            pl.BlockSpec((_BR, EW), lambda i: (i, 0)),
            pl.BlockSpec((1, HID), lambda i: (0, 0)),
            pl.BlockSpec((HID, HID), lambda i: (0, 0)),
            pl.BlockSpec((_BR, EW), lambda i: (i, 0)),
        ],
        out_specs=_q_specs,
        out_shape=_q_shapes,
    )


_k_mid_e = _make_mid(N_EDGES)
_k_mid_n = _make_mid(N_NODES)


def _fin_body(*refs):
    a = refs[:NQ]
    d_ref, b_ref, wl_ref, bl_ref, o_ref, m_ref = refs[NQ:]
    i = pl.program_id(0)

    @pl.when(i == 0)
    def _():
        m_ref[...] = jnp.full((8, HID), -jnp.inf, jnp.float32)

    dinv = 1.0 / jnp.maximum(d_ref[:, 0:1], 1e-12)
    x = jnp.concatenate([r[...] for r in a], axis=1)
    x1 = jax.nn.sigmoid(x * dinv + b_ref[...])
    bm = jnp.max(x1, axis=0, keepdims=True)
    m_ref[0:1, :] = jnp.maximum(m_ref[0:1, :], bm)

    @pl.when(i == N_NODES // _BR - 1)
    def _():
        o_ref[...] = jnp.dot(m_ref[0:1, :], wl_ref[...],
                             preferred_element_type=jnp.float32,
                             precision=lax.Precision.HIGHEST) + bl_ref[...]


_k_fin = pl.pallas_call(
    _fin_body,
    grid=(N_NODES // _BR,),
    in_specs=[
        *(pl.BlockSpec((_BR, EW), lambda i: (i, 0)) for _ in range(NQ)),
        pl.BlockSpec((_BR, EW), lambda i: (i, 0)),
        pl.BlockSpec((1, HID), lambda i: (0, 0)),
        pl.BlockSpec((HID, 1), lambda i: (0, 0)),
        pl.BlockSpec((1, 1), lambda i: (0, 0)),
    ],
    out_specs=pl.BlockSpec((1, 1), lambda i: (0, 0)),
    out_shape=_f32((1, 1)),
    scratch_shapes=[pltpu.VMEM((8, HID), jnp.float32)],
)


def kernel(x_0, node_idx, edge_idx, W01_0, b1_0, W10_0, b0_0,
           W01_1, b1_1, W10_1, b0_1, W_lin, b_lin):
    zeros16 = jnp.zeros((OUT_SLICE, EW), jnp.float32)
    ones_n = jnp.ones((N_NODES, EW), jnp.float32)
    ones_e = jnp.ones((N_EDGES, EW), jnp.float32)

    ecnt, ncnt = _k_hist(ones_n, ones_e, node_idx, edge_idx, zeros16)
    te, tn = _k_norm(ecnt, ncnt)

    xb = _k_in(x_0, W01_0, tn)
    *aa, d1s, d0s = _k_pass_norm(*xb, tn, te, node_idx, edge_idx, zeros16)
    xe = _k_mid_e(*aa, d1s, b1_0.reshape(1, -1), W10_0, te)
    ab = _k_pass(*xe, edge_idx, node_idx, zeros16)
    xb = _k_mid_n(*ab, d0s, b0_0.reshape(1, -1), W01_1, tn)
    aa = _k_pass(*xb, node_idx, edge_idx, zeros16)
    xe = _k_mid_e(*aa, d1s, b1_1.reshape(1, -1), W10_1, te)
    ab = _k_pass(*xe, edge_idx, node_idx, zeros16)

    out = _k_fin(*ab, d0s, b0_1.reshape(1, -1), W_lin, b_lin.reshape(1, 1))
    return out.reshape(1)


# speedup vs baseline: 1.1441x; 1.1441x over previous
import functools

import jax
import jax.numpy as jnp
from jax import lax
from jax.experimental import pallas as pl
from jax.experimental.pallas import tpu as pltpu
from jax.experimental.pallas import tpu_sc as plsc

N_NODES = 10000
N_EDGES = 20000
NNZ = 320000
HID = 128
QW = 32

NS = 16
PER_W = NNZ // NS
CHUNK = 1000
ITERS = PER_W // CHUNK
CHUNK_P = 1000
ITERS_P = PER_W // CHUNK_P
OUT_SLICE = 2000

_mesh = plsc.VectorSubcoreMesh(core_axis_name="c", subcore_axis_name="s")

_SC_PARAMS = pltpu.CompilerParams(use_tc_tiling_on_sc=False)


def _f32(shape):
    return jax.ShapeDtypeStruct(shape, jnp.float32)


def _chain(x, token):
    return lax.optimization_barrier((x, token))[0]


def _zero_acc(acc, zeros_hbm, s):
    @pl.when(s < N_EDGES // OUT_SLICE)
    def _():
        pltpu.sync_copy(zeros_hbm, acc.at[pl.ds(s * OUT_SLICE, OUT_SLICE)])


def _copy_out(acc, out, s, cond):
    @pl.when(cond)
    def _():
        sl = pl.ds(s * OUT_SLICE, OUT_SLICE)
        pltpu.sync_copy(acc.at[sl], out.at[sl])


def _load_idx2d(idx_hbm, buf, s, chunk, iters):
    base = s * PER_W

    @pl.loop(0, iters)
    def _(i):
        pltpu.sync_copy(idx_hbm.at[pl.ds(base + i * chunk, chunk)], buf.at[i])


def _run_phase(tbl, acc, src2d, dst2d, rows0, rows1, sem0, sem1, iters):
    @pl.loop(0, iters)
    def _(i):
        pltpu.sync_copy(tbl.at[src2d.at[i]], rows0)
        pltpu.sync_copy(rows0, acc.at[dst2d.at[i]], add=True)


def _sc_scratch(width, chunk, iters):
    return [
        pltpu.VMEM((iters, chunk), jnp.int32),
        pltpu.VMEM((iters, chunk), jnp.int32),
        pltpu.VMEM((chunk, width), jnp.float32),
        pltpu.VMEM((chunk, width), jnp.float32),
        pltpu.VMEM_SHARED((N_EDGES, width), jnp.float32),
        pltpu.SemaphoreType.DMA,
        pltpu.SemaphoreType.DMA,
    ]


@functools.partial(
    pl.kernel,
    out_type=(_f32((N_EDGES, 16)), _f32((N_NODES, 16))),
    mesh=_mesh,
    compiler_params=_SC_PARAMS,
    scratch_types=_sc_scratch(16, CHUNK, ITERS),
)
def _k_scalar_sums(tbl_n, tbl_e, nidx, eidx, zeros_hbm, oute, outn,
                   buf_n, buf_e, rows0, rows1, acc, sem0, sem1):
    c = lax.axis_index("c")
    s = lax.axis_index("s")

    _zero_acc(acc, zeros_hbm, s)
    _load_idx2d(nidx, buf_n, s, CHUNK, ITERS)
    _load_idx2d(eidx, buf_e, s, CHUNK, ITERS)
    plsc.subcore_barrier()

    pl.when(c == 0)(lambda: _run_phase(
        tbl_n, acc, buf_n, buf_e, rows0, rows1, sem0, sem1, ITERS))
    pl.when(c == 1)(lambda: _run_phase(
        tbl_e, acc, buf_e, buf_n, rows0, rows1, sem0, sem1, ITERS))
    plsc.subcore_barrier()

    _copy_out(acc, oute, s, jnp.logical_and(c == 0, s < N_EDGES // OUT_SLICE))
    _copy_out(acc, outn, s, jnp.logical_and(c == 1, s < N_NODES // OUT_SLICE))


@functools.partial(
    pl.kernel,
    out_type=(_f32((N_EDGES, QW)), _f32((N_EDGES, QW))),
    mesh=_mesh,
    compiler_params=_SC_PARAMS,
    scratch_types=_sc_scratch(QW, CHUNK_P, ITERS_P),
)
def _k_pass(tbl_lo, tbl_hi, sidx, didx, zeros_hbm, out_lo, out_hi,
            buf_s, buf_d, rows0, rows1, acc, sem0, sem1):
    c = lax.axis_index("c")
    s = lax.axis_index("s")

    _zero_acc(acc, zeros_hbm, s)
    _load_idx2d(sidx, buf_s, s, CHUNK_P, ITERS_P)
    _load_idx2d(didx, buf_d, s, CHUNK_P, ITERS_P)
    plsc.subcore_barrier()

    pl.when(c == 0)(lambda: _run_phase(
        tbl_lo, acc, buf_s, buf_d, rows0, rows1, sem0, sem1, ITERS_P))
    pl.when(c == 1)(lambda: _run_phase(
        tbl_hi, acc, buf_s, buf_d, rows0, rows1, sem0, sem1, ITERS_P))
    plsc.subcore_barrier()

    _copy_out(acc, out_lo, s, jnp.logical_and(c == 0, s < N_EDGES // OUT_SLICE))
    _copy_out(acc, out_hi, s, jnp.logical_and(c == 1, s < N_EDGES // OUT_SLICE))


def _norm_body(ec_ref, nc_ref, te_ref, tn_ref):
    r = lax.rsqrt(jnp.maximum(ec_ref[...], 1.0))
    te_ref[...] = r * r * r
    tn_ref[...] = lax.rsqrt(jnp.maximum(nc_ref[...], 1.0))


_k_norm = pl.pallas_call(
    _norm_body,
    out_shape=(_f32((N_EDGES, 16)), _f32((N_NODES, 16))),
)


_BR = 2000


def _write_quarters(y, out_refs):
    for j, o_ref in enumerate(out_refs):
        o_ref[...] = y[:, j * QW:(j + 1) * QW]


def _in_body(x_ref, w_ref, t_ref, *out_refs):
    y = jnp.dot(x_ref[...], w_ref[...], preferred_element_type=jnp.float32,
                precision=lax.Precision.HIGHEST)
    _write_quarters(y * t_ref[:, 0:1], out_refs)


_h_specs = tuple(pl.BlockSpec((_BR, QW), lambda i: (i, 0)) for _ in range(4))
_h_shapes = tuple(_f32((N_EDGES, QW)) for _ in range(4))

_k_in = pl.pallas_call(
    _in_body,
    grid=(N_NODES // _BR,),
    in_specs=[
        pl.BlockSpec((_BR, HID), lambda i: (i, 0)),
        pl.BlockSpec((HID, HID), lambda i: (0, 0)),
        pl.BlockSpec((_BR, 16), lambda i: (i, 0)),
    ],
    out_specs=_h_specs,
    out_shape=_h_shapes,
)


def _mid_body(a0_ref, a1_ref, a2_ref, a3_ref, d_ref, b_ref, w_ref, t_ref,
              *out_refs):
    dinv = 1.0 / jnp.maximum(d_ref[:, 0:1], 1e-12)
    x = jnp.concatenate(
        [a0_ref[...], a1_ref[...], a2_ref[...], a3_ref[...]], axis=1)
    x1 = jax.nn.sigmoid(x * dinv + b_ref[...])
    y = jnp.dot(x1, w_ref[...], preferred_element_type=jnp.float32,
                precision=lax.Precision.HIGHEST)
    _write_quarters(y * t_ref[:, 0:1], out_refs)


def _make_mid(n_rows):
    return pl.pallas_call(
        _mid_body,
        grid=(n_rows // _BR,),
        in_specs=[
            *(pl.BlockSpec((_BR, QW), lambda i: (i, 0)) for _ in range(4)),
            pl.BlockSpec((_BR, 16), lambda i: (i, 0)),
            pl.BlockSpec((1, HID), lambda i: (0, 0)),
            pl.BlockSpec((HID, HID), lambda i: (0, 0)),
            pl.BlockSpec((_BR, 16), lambda i: (i, 0)),
        ],
        out_specs=_h_specs,
        out_shape=_h_shapes,
    )


_k_mid_e = _make_mid(N_EDGES)
_k_mid_n = _make_mid(N_NODES)


def _fin_body(a0_ref, a1_ref, a2_ref, a3_ref, d_ref, b_ref, wl_ref, bl_ref,
              o_ref, m_ref):
    i = pl.program_id(0)

    @pl.when(i == 0)
    def _():
        m_ref[...] = jnp.full((8, HID), -jnp.inf, jnp.float32)

    dinv = 1.0 / jnp.maximum(d_ref[:, 0:1], 1e-12)
    x = jnp.concatenate(
        [a0_ref[...], a1_ref[...], a2_ref[...], a3_ref[...]], axis=1)
    x1 = jax.nn.sigmoid(x * dinv + b_ref[...])
    bm = jnp.max(x1, axis=0, keepdims=True)
    m_ref[0:1, :] = jnp.maximum(m_ref[0:1, :], bm)

    @pl.when(i == N_NODES // _BR - 1)
    def _():
        o_ref[...] = jnp.dot(m_ref[0:1, :], wl_ref[...],
                             preferred_element_type=jnp.float32,
                             precision=lax.Precision.HIGHEST) + bl_ref[...]


_k_fin = pl.pallas_call(
    _fin_body,
    grid=(N_NODES // _BR,),
    in_specs=[
        *(pl.BlockSpec((_BR, QW), lambda i: (i, 0)) for _ in range(4)),
        pl.BlockSpec((_BR, 16), lambda i: (i, 0)),
        pl.BlockSpec((1, HID), lambda i: (0, 0)),
        pl.BlockSpec((HID, 1), lambda i: (0, 0)),
        pl.BlockSpec((1, 1), lambda i: (0, 0)),
    ],
    out_specs=pl.BlockSpec((1, 1), lambda i: (0, 0)),
    out_shape=_f32((1, 1)),
    scratch_shapes=[pltpu.VMEM((8, HID), jnp.float32)],
)


def kernel(x_0, node_idx, edge_idx, W01_0, b1_0, W10_0, b0_0,
           W01_1, b1_1, W10_1, b0_1, W_lin, b_lin):
    zeros16 = jnp.zeros((OUT_SLICE, 16), jnp.float32)
    zeros32 = jnp.zeros((OUT_SLICE, QW), jnp.float32)
    ones_n = jnp.ones((N_NODES, 16), jnp.float32)
    ones_e = jnp.ones((N_EDGES, 16), jnp.float32)

    ecnt, ncnt = _k_scalar_sums(ones_n, ones_e, node_idx, edge_idx, zeros16)
    te, tn = _k_norm(ecnt, ncnt)
    d1s, d0s = _k_scalar_sums(_chain(tn, ecnt), te, node_idx, edge_idx,
                              zeros16)

    def seg_pass(q, sidx, didx, token):
        a0, a1 = _k_pass(_chain(q[0], token), q[1], sidx, didx, zeros32)
        a2, a3 = _k_pass(_chain(q[2], a0), q[3], sidx, didx, zeros32)
        return a0, a1, a2, a3

    xb = _k_in(x_0, W01_0, tn)
    aa = seg_pass(xb, node_idx, edge_idx, d0s)
    xe = _k_mid_e(*aa, d1s, b1_0.reshape(1, -1), W10_0, te)
    ab = seg_pass(xe, edge_idx, node_idx, aa[2])
    xb = _k_mid_n(*ab, d0s, b0_0.reshape(1, -1), W01_1, tn)
    aa = seg_pass(xb, node_idx, edge_idx, ab[2])
    xe = _k_mid_e(*aa, d1s, b1_1.reshape(1, -1), W10_1, te)
    ab = seg_pass(xe, edge_idx, node_idx, aa[2])

    out = _k_fin(*ab, d0s, b0_1.reshape(1, -1), W_lin, b_lin.reshape(1, 1))
    return out.reshape(1)


# speedup vs baseline: 1.2008x; 1.0496x over previous
import functools

import jax
import jax.numpy as jnp
from jax import lax
from jax.experimental import pallas as pl
from jax.experimental.pallas import tpu as pltpu
from jax.experimental.pallas import tpu_sc as plsc

N_NODES = 10000
N_EDGES = 20000
NNZ = 320000
HID = 128
QW = 32

NS = 16
PER_W = NNZ // NS
CHUNK = 1000
ITERS = PER_W // CHUNK
CHUNK_P = 1000
ITERS_P = PER_W // CHUNK_P
OUT_SLICE = 2000

_mesh = plsc.VectorSubcoreMesh(core_axis_name="c", subcore_axis_name="s")

_SC_PARAMS = pltpu.CompilerParams(use_tc_tiling_on_sc=False)


def _f32(shape):
    return jax.ShapeDtypeStruct(shape, jnp.float32)


def _chain(x, token):
    return lax.optimization_barrier((x, token))[0]


def _zero_acc(acc, zeros_hbm, s):
    @pl.when(s < N_EDGES // OUT_SLICE)
    def _():
        pltpu.sync_copy(zeros_hbm, acc.at[pl.ds(s * OUT_SLICE, OUT_SLICE)])


def _copy_out(acc, out, s, cond):
    @pl.when(cond)
    def _():
        sl = pl.ds(s * OUT_SLICE, OUT_SLICE)
        pltpu.sync_copy(acc.at[sl], out.at[sl])


def _load_idx2d(idx_hbm, buf, s, chunk, iters):
    base = s * PER_W

    @pl.loop(0, iters)
    def _(i):
        pltpu.sync_copy(idx_hbm.at[pl.ds(base + i * chunk, chunk)], buf.at[i])


def _run_phase(tbl, acc, src2d, dst2d, rows0, rows1, sem0, sem1, iters):
    @pl.loop(0, iters)
    def _(i):
        pltpu.sync_copy(tbl.at[src2d.at[i]], rows0)
        pltpu.sync_copy(rows0, acc.at[dst2d.at[i]], add=True)


def _sc_scratch(width, chunk, iters, acc_rows=N_EDGES):
    return [
        pltpu.VMEM((iters, chunk), jnp.int32),
        pltpu.VMEM((iters, chunk), jnp.int32),
        pltpu.VMEM((chunk, width), jnp.float32),
        pltpu.VMEM((chunk, width), jnp.float32),
        pltpu.VMEM_SHARED((acc_rows, width), jnp.float32),
        pltpu.SemaphoreType.DMA,
        pltpu.SemaphoreType.DMA,
    ]


@functools.partial(
    pl.kernel,
    out_type=(_f32((N_EDGES, 16)), _f32((N_NODES, 16))),
    mesh=_mesh,
    compiler_params=_SC_PARAMS,
    scratch_types=_sc_scratch(16, CHUNK, ITERS),
)
def _k_scalar_sums(tbl_n, tbl_e, nidx, eidx, zeros_hbm, oute, outn,
                   buf_n, buf_e, rows0, rows1, acc, sem0, sem1):
    c = lax.axis_index("c")
    s = lax.axis_index("s")

    _zero_acc(acc, zeros_hbm, s)
    _load_idx2d(nidx, buf_n, s, CHUNK, ITERS)
    _load_idx2d(eidx, buf_e, s, CHUNK, ITERS)
    plsc.subcore_barrier()

    pl.when(c == 0)(lambda: _run_phase(
        tbl_n, acc, buf_n, buf_e, rows0, rows1, sem0, sem1, ITERS))
    pl.when(c == 1)(lambda: _run_phase(
        tbl_e, acc, buf_e, buf_n, rows0, rows1, sem0, sem1, ITERS))
    plsc.subcore_barrier()

    _copy_out(acc, oute, s, jnp.logical_and(c == 0, s < N_EDGES // OUT_SLICE))
    _copy_out(acc, outn, s, jnp.logical_and(c == 1, s < N_NODES // OUT_SLICE))


@functools.partial(
    pl.kernel,
    out_type=(_f32((N_EDGES, QW)), _f32((N_EDGES, QW))),
    mesh=_mesh,
    compiler_params=_SC_PARAMS,
    scratch_types=_sc_scratch(QW, CHUNK_P, ITERS_P),
)
def _k_pass(tbl_lo, tbl_hi, sidx, didx, zeros_hbm, out_lo, out_hi,
            buf_s, buf_d, rows0, rows1, acc, sem0, sem1):
    c = lax.axis_index("c")
    s = lax.axis_index("s")

    _zero_acc(acc, zeros_hbm, s)
    _load_idx2d(sidx, buf_s, s, CHUNK_P, ITERS_P)
    _load_idx2d(didx, buf_d, s, CHUNK_P, ITERS_P)
    plsc.subcore_barrier()

    pl.when(c == 0)(lambda: _run_phase(
        tbl_lo, acc, buf_s, buf_d, rows0, rows1, sem0, sem1, ITERS_P))
    pl.when(c == 1)(lambda: _run_phase(
        tbl_hi, acc, buf_s, buf_d, rows0, rows1, sem0, sem1, ITERS_P))
    plsc.subcore_barrier()

    _copy_out(acc, out_lo, s, jnp.logical_and(c == 0, s < N_EDGES // OUT_SLICE))
    _copy_out(acc, out_hi, s, jnp.logical_and(c == 1, s < N_EDGES // OUT_SLICE))


@functools.partial(
    pl.kernel,
    out_type=(_f32((N_NODES, 64)), _f32((N_NODES, 64))),
    mesh=_mesh,
    compiler_params=_SC_PARAMS,
    scratch_types=_sc_scratch(64, 400, PER_W // 400, N_NODES),
)
def _k_pass_n(tbl_lo, tbl_hi, sidx, didx, zeros_hbm, out_lo, out_hi,
              buf_s, buf_d, rows0, rows1, acc, sem0, sem1):
    c = lax.axis_index("c")
    s = lax.axis_index("s")
    it = PER_W // 400

    @pl.when(s < N_NODES // OUT_SLICE)
    def _():
        pltpu.sync_copy(zeros_hbm, acc.at[pl.ds(s * OUT_SLICE, OUT_SLICE)])
    _load_idx2d(sidx, buf_s, s, 400, it)
    _load_idx2d(didx, buf_d, s, 400, it)
    plsc.subcore_barrier()

    pl.when(c == 0)(lambda: _run_phase(
        tbl_lo, acc, buf_s, buf_d, rows0, rows1, sem0, sem1, it))
    pl.when(c == 1)(lambda: _run_phase(
        tbl_hi, acc, buf_s, buf_d, rows0, rows1, sem0, sem1, it))
    plsc.subcore_barrier()

    _copy_out(acc, out_lo, s, jnp.logical_and(c == 0, s < N_NODES // OUT_SLICE))
    _copy_out(acc, out_hi, s, jnp.logical_and(c == 1, s < N_NODES // OUT_SLICE))


def _norm_body(ec_ref, nc_ref, te_ref, tn_ref):
    r = lax.rsqrt(jnp.maximum(ec_ref[...], 1.0))
    te_ref[...] = r * r * r
    tn_ref[...] = lax.rsqrt(jnp.maximum(nc_ref[...], 1.0))


_k_norm = pl.pallas_call(
    _norm_body,
    out_shape=(_f32((N_EDGES, 16)), _f32((N_NODES, 16))),
)


_BR = 2000


def _write_quarters(y, out_refs):
    for j, o_ref in enumerate(out_refs):
        o_ref[...] = y[:, j * QW:(j + 1) * QW]


def _in_body(x_ref, w_ref, t_ref, *out_refs):
    y = jnp.dot(x_ref[...], w_ref[...], preferred_element_type=jnp.float32,
                precision=lax.Precision.HIGHEST)
    _write_quarters(y * t_ref[:, 0:1], out_refs)


_h_specs = tuple(pl.BlockSpec((_BR, QW), lambda i: (i, 0)) for _ in range(4))
_h_shapes = tuple(_f32((N_EDGES, QW)) for _ in range(4))

_k_in = pl.pallas_call(
    _in_body,
    grid=(N_NODES // _BR,),
    in_specs=[
        pl.BlockSpec((_BR, HID), lambda i: (i, 0)),
        pl.BlockSpec((HID, HID), lambda i: (0, 0)),
        pl.BlockSpec((_BR, 16), lambda i: (i, 0)),
    ],
    out_specs=_h_specs,
    out_shape=_h_shapes,
)


def _make_mid(n_rows, in_widths, out_widths, out_rows):
    def body(*refs):
        a = refs[:len(in_widths)]
        d_ref, b_ref, w_ref, t_ref = refs[len(in_widths):len(in_widths) + 4]
        out_refs = refs[len(in_widths) + 4:]
        dinv = 1.0 / jnp.maximum(d_ref[:, 0:1], 1e-12)
        x = jnp.concatenate([r[...] for r in a], axis=1)
        x1 = jax.nn.sigmoid(x * dinv + b_ref[...])
        y = jnp.dot(x1, w_ref[...], preferred_element_type=jnp.float32,
                    precision=lax.Precision.HIGHEST)
        off = 0
        for w, o_ref in zip(out_widths, out_refs):
            o_ref[...] = y[:, off:off + w]
            off += w

    return pl.pallas_call(
        body,
        grid=(n_rows // _BR,),
        in_specs=[
            *(pl.BlockSpec((_BR, w), lambda i: (i, 0)) for w in in_widths),
            pl.BlockSpec((_BR, 16), lambda i: (i, 0)),
            pl.BlockSpec((1, HID), lambda i: (0, 0)),
            pl.BlockSpec((HID, HID), lambda i: (0, 0)),
            pl.BlockSpec((_BR, 16), lambda i: (i, 0)),
        ],
        out_specs=tuple(pl.BlockSpec((_BR, w), lambda i: (i, 0))
                        for w in out_widths),
        out_shape=tuple(_f32((out_rows, w)) for w in out_widths),
    )


_k_mid_e = _make_mid(N_EDGES, (QW,) * 4, (64, 64), N_EDGES)
_k_mid_n = _make_mid(N_NODES, (64, 64), (QW,) * 4, N_EDGES)


def _fin_body(lo_ref, hi_ref, d_ref, b_ref, wl_ref, bl_ref, o_ref, m_ref):
    i = pl.program_id(0)

    @pl.when(i == 0)
    def _():
        m_ref[...] = jnp.full((8, HID), -jnp.inf, jnp.float32)

    dinv = 1.0 / jnp.maximum(d_ref[:, 0:1], 1e-12)
    x = jnp.concatenate([lo_ref[...], hi_ref[...]], axis=1)
    x1 = jax.nn.sigmoid(x * dinv + b_ref[...])
    bm = jnp.max(x1, axis=0, keepdims=True)
    m_ref[0:1, :] = jnp.maximum(m_ref[0:1, :], bm)

    @pl.when(i == N_NODES // _BR - 1)
    def _():
        o_ref[...] = jnp.dot(m_ref[0:1, :], wl_ref[...],
                             preferred_element_type=jnp.float32,
                             precision=lax.Precision.HIGHEST) + bl_ref[...]


_k_fin = pl.pallas_call(
    _fin_body,
    grid=(N_NODES // _BR,),
    in_specs=[
        pl.BlockSpec((_BR, 64), lambda i: (i, 0)),
        pl.BlockSpec((_BR, 64), lambda i: (i, 0)),
        pl.BlockSpec((_BR, 16), lambda i: (i, 0)),
        pl.BlockSpec((1, HID), lambda i: (0, 0)),
        pl.BlockSpec((HID, 1), lambda i: (0, 0)),
        pl.BlockSpec((1, 1), lambda i: (0, 0)),
    ],
    out_specs=pl.BlockSpec((1, 1), lambda i: (0, 0)),
    out_shape=_f32((1, 1)),
    scratch_shapes=[pltpu.VMEM((8, HID), jnp.float32)],
)


def kernel(x_0, node_idx, edge_idx, W01_0, b1_0, W10_0, b0_0,
           W01_1, b1_1, W10_1, b0_1, W_lin, b_lin):
    zeros16 = jnp.zeros((OUT_SLICE, 16), jnp.float32)
    zeros32 = jnp.zeros((OUT_SLICE, QW), jnp.float32)
    one = jnp.sum(b_lin) * 0.0 + 1.0
    ones_n = jnp.full((N_NODES, 16), 1.0, jnp.float32) * one
    ones_e = jnp.full((N_EDGES, 16), 1.0, jnp.float32) * one

    ecnt, ncnt = _k_scalar_sums(ones_n, ones_e, node_idx, edge_idx, zeros16)
    te, tn = _k_norm(ecnt, ncnt)
    d1s, d0s = _k_scalar_sums(_chain(tn, ecnt), te, node_idx, edge_idx,
                              zeros16)

    zeros64 = jnp.zeros((OUT_SLICE, 64), jnp.float32)

    def seg_pass_e(q, token):
        a0, a1 = _k_pass(_chain(q[0], token), q[1], node_idx, edge_idx,
                         zeros32)
        a2, a3 = _k_pass(_chain(q[2], a0), q[3], node_idx, edge_idx, zeros32)
        return a0, a1, a2, a3

    xb = _k_in(x_0, W01_0, tn)
    aa = seg_pass_e(xb, d0s)
    xe = _k_mid_e(*aa, d1s, b1_0.reshape(1, -1), W10_0, te)
    ab = _k_pass_n(_chain(xe[0], aa[2]), xe[1], edge_idx, node_idx, zeros64)
    xb = _k_mid_n(*ab, d0s, b0_0.reshape(1, -1), W01_1, tn)
    aa = seg_pass_e(xb, ab[0])
    xe = _k_mid_e(*aa, d1s, b1_1.reshape(1, -1), W10_1, te)
    ab = _k_pass_n(_chain(xe[0], aa[2]), xe[1], edge_idx, node_idx, zeros64)

    out = _k_fin(*ab, d0s, b0_1.reshape(1, -1), W_lin, b_lin.reshape(1, 1))
    return out.reshape(1)


# speedup vs baseline: 1.2016x; 1.0007x over previous
import functools

import jax
import jax.numpy as jnp
from jax import lax
from jax.experimental import pallas as pl
from jax.experimental.pallas import tpu as pltpu
from jax.experimental.pallas import tpu_sc as plsc

N_NODES = 10000
N_EDGES = 20000
NNZ = 320000
HID = 128
QW = 32

NS = 16
PER_W = NNZ // NS
CHUNK = 1000
ITERS = PER_W // CHUNK
CHUNK_P = 1000
ITERS_P = PER_W // CHUNK_P
OUT_SLICE = 2000

_mesh = plsc.VectorSubcoreMesh(core_axis_name="c", subcore_axis_name="s")

_SC_PARAMS = pltpu.CompilerParams(use_tc_tiling_on_sc=False)


def _f32(shape):
    return jax.ShapeDtypeStruct(shape, jnp.float32)


def _chain(x, token):
    return lax.optimization_barrier((x, token))[0]


def _zero_acc(acc, zeros_hbm, s):
    @pl.when(s < N_EDGES // OUT_SLICE)
    def _():
        pltpu.sync_copy(zeros_hbm, acc.at[pl.ds(s * OUT_SLICE, OUT_SLICE)])


def _copy_out(acc, out, s, cond):
    @pl.when(cond)
    def _():
        sl = pl.ds(s * OUT_SLICE, OUT_SLICE)
        pltpu.sync_copy(acc.at[sl], out.at[sl])


def _load_idx2d(idx_hbm, buf, s, chunk, iters):
    base = s * PER_W

    @pl.loop(0, iters)
    def _(i):
        pltpu.sync_copy(idx_hbm.at[pl.ds(base + i * chunk, chunk)], buf.at[i])


def _run_phase(tbl, acc, src2d, dst2d, rows0, rows1, sem0, sem1, iters):
    @pl.loop(0, iters)
    def _(i):
        pltpu.sync_copy(tbl.at[src2d.at[i]], rows0)
        pltpu.sync_copy(rows0, acc.at[dst2d.at[i]], add=True)


def _sc_scratch(width, chunk, iters, acc_rows=N_EDGES):
    return [
        pltpu.VMEM((iters, chunk), jnp.int32),
        pltpu.VMEM((iters, chunk), jnp.int32),
        pltpu.VMEM((chunk, width), jnp.float32),
        pltpu.VMEM((chunk, width), jnp.float32),
        pltpu.VMEM_SHARED((acc_rows, width), jnp.float32),
        pltpu.SemaphoreType.DMA,
        pltpu.SemaphoreType.DMA,
    ]


@functools.partial(
    pl.kernel,
    out_type=(_f32((N_EDGES, 16)), _f32((N_NODES, 16))),
    mesh=_mesh,
    compiler_params=_SC_PARAMS,
    scratch_types=_sc_scratch(16, CHUNK, ITERS),
)
def _k_scalar_sums(tbl_n, tbl_e, nidx, eidx, zeros_hbm, oute, outn,
                   buf_n, buf_e, rows0, rows1, acc, sem0, sem1):
    c = lax.axis_index("c")
    s = lax.axis_index("s")

    _zero_acc(acc, zeros_hbm, s)
    _load_idx2d(nidx, buf_n, s, CHUNK, ITERS)
    _load_idx2d(eidx, buf_e, s, CHUNK, ITERS)
    plsc.subcore_barrier()

    pl.when(c == 0)(lambda: _run_phase(
        tbl_n, acc, buf_n, buf_e, rows0, rows1, sem0, sem1, ITERS))
    pl.when(c == 1)(lambda: _run_phase(
        tbl_e, acc, buf_e, buf_n, rows0, rows1, sem0, sem1, ITERS))
    plsc.subcore_barrier()

    _copy_out(acc, oute, s, jnp.logical_and(c == 0, s < N_EDGES // OUT_SLICE))
    _copy_out(acc, outn, s, jnp.logical_and(c == 1, s < N_NODES // OUT_SLICE))


@functools.partial(
    pl.kernel,
    out_type=(_f32((N_EDGES, QW)), _f32((N_EDGES, QW))),
    mesh=_mesh,
    compiler_params=_SC_PARAMS,
    scratch_types=_sc_scratch(QW, CHUNK_P, ITERS_P),
)
def _k_pass(tbl_lo, tbl_hi, sidx, didx, zeros_hbm, out_lo, out_hi,
            buf_s, buf_d, rows0, rows1, acc, sem0, sem1):
    c = lax.axis_index("c")
    s = lax.axis_index("s")

    _zero_acc(acc, zeros_hbm, s)
    _load_idx2d(sidx, buf_s, s, CHUNK_P, ITERS_P)
    _load_idx2d(didx, buf_d, s, CHUNK_P, ITERS_P)
    plsc.subcore_barrier()

    pl.when(c == 0)(lambda: _run_phase(
        tbl_lo, acc, buf_s, buf_d, rows0, rows1, sem0, sem1, ITERS_P))
    pl.when(c == 1)(lambda: _run_phase(
        tbl_hi, acc, buf_s, buf_d, rows0, rows1, sem0, sem1, ITERS_P))
    plsc.subcore_barrier()

    _copy_out(acc, out_lo, s, jnp.logical_and(c == 0, s < N_EDGES // OUT_SLICE))
    _copy_out(acc, out_hi, s, jnp.logical_and(c == 1, s < N_EDGES // OUT_SLICE))


@functools.partial(
    pl.kernel,
    out_type=(_f32((N_NODES, 64)), _f32((N_NODES, 64))),
    mesh=_mesh,
    compiler_params=_SC_PARAMS,
    scratch_types=_sc_scratch(64, 400, PER_W // 400, N_NODES),
)
def _k_pass_n(tbl_lo, tbl_hi, sidx, didx, zeros_hbm, out_lo, out_hi,
              buf_s, buf_d, rows0, rows1, acc, sem0, sem1):
    c = lax.axis_index("c")
    s = lax.axis_index("s")
    it = PER_W // 400

    @pl.when(s < N_NODES // OUT_SLICE)
    def _():
        pltpu.sync_copy(zeros_hbm, acc.at[pl.ds(s * OUT_SLICE, OUT_SLICE)])
    _load_idx2d(sidx, buf_s, s, 400, it)
    _load_idx2d(didx, buf_d, s, 400, it)
    plsc.subcore_barrier()

    pl.when(c == 0)(lambda: _run_phase(
        tbl_lo, acc, buf_s, buf_d, rows0, rows1, sem0, sem1, it))
    pl.when(c == 1)(lambda: _run_phase(
        tbl_hi, acc, buf_s, buf_d, rows0, rows1, sem0, sem1, it))
    plsc.subcore_barrier()

    _copy_out(acc, out_lo, s, jnp.logical_and(c == 0, s < N_NODES // OUT_SLICE))
    _copy_out(acc, out_hi, s, jnp.logical_and(c == 1, s < N_NODES // OUT_SLICE))


def _norm_body(ec_ref, nc_ref, te_ref, tn_ref):
    r = lax.rsqrt(jnp.maximum(ec_ref[...], 1.0))
    te_ref[...] = r * r * r
    tn_ref[...] = lax.rsqrt(jnp.maximum(nc_ref[...], 1.0))


_k_norm = pl.pallas_call(
    _norm_body,
    out_shape=(_f32((N_EDGES, 16)), _f32((N_NODES, 16))),
)


_BR = 2000


def _write_quarters(y, out_refs):
    for j, o_ref in enumerate(out_refs):
        o_ref[...] = y[:, j * QW:(j + 1) * QW]


def _in_body(x_ref, w_ref, t_ref, *out_refs):
    y = jnp.dot(x_ref[...], w_ref[...], preferred_element_type=jnp.float32,
                precision=lax.Precision.HIGHEST)
    _write_quarters(y * t_ref[:, 0:1], out_refs)


_h_specs = tuple(pl.BlockSpec((_BR, QW), lambda i: (i, 0)) for _ in range(4))
_h_shapes = tuple(_f32((N_EDGES, QW)) for _ in range(4))

_k_in = pl.pallas_call(
    _in_body,
    grid=(N_NODES // _BR,),
    in_specs=[
        pl.BlockSpec((_BR, HID), lambda i: (i, 0)),
        pl.BlockSpec((HID, HID), lambda i: (0, 0)),
        pl.BlockSpec((_BR, 16), lambda i: (i, 0)),
    ],
    out_specs=_h_specs,
    out_shape=_h_shapes,
)


def _make_mid(n_rows, in_widths, out_widths, out_rows):
    def body(*refs):
        a = refs[:len(in_widths)]
        d_ref, b_ref, w_ref, t_ref = refs[len(in_widths):len(in_widths) + 4]
        out_refs = refs[len(in_widths) + 4:]
        dinv = 1.0 / jnp.maximum(d_ref[:, 0:1], 1e-12)
        x = jnp.concatenate([r[...] for r in a], axis=1)
        x1 = jax.nn.sigmoid(x * dinv + b_ref[...])
        y = jnp.dot(x1, w_ref[...], preferred_element_type=jnp.float32,
                    precision=lax.Precision.HIGHEST)
        y = y * t_ref[:, 0:1]
        off = 0
        for w, o_ref in zip(out_widths, out_refs):
            o_ref[...] = y[:, off:off + w]
            off += w

    return pl.pallas_call(
        body,
        grid=(n_rows // _BR,),
        in_specs=[
            *(pl.BlockSpec((_BR, w), lambda i: (i, 0)) for w in in_widths),
            pl.BlockSpec((_BR, 16), lambda i: (i, 0)),
            pl.BlockSpec((1, HID), lambda i: (0, 0)),
            pl.BlockSpec((HID, HID), lambda i: (0, 0)),
            pl.BlockSpec((_BR, 16), lambda i: (i, 0)),
        ],
        out_specs=tuple(pl.BlockSpec((_BR, w), lambda i: (i, 0))
                        for w in out_widths),
        out_shape=tuple(_f32((out_rows, w)) for w in out_widths),
    )


_k_mid_e = _make_mid(N_EDGES, (QW,) * 4, (64, 64), N_EDGES)
_k_mid_n = _make_mid(N_NODES, (64, 64), (QW,) * 4, N_EDGES)


def _fin_body(lo_ref, hi_ref, d_ref, b_ref, wl_ref, bl_ref, o_ref, m_ref):
    i = pl.program_id(0)

    @pl.when(i == 0)
    def _():
        m_ref[...] = jnp.full((8, HID), -jnp.inf, jnp.float32)

    dinv = 1.0 / jnp.maximum(d_ref[:, 0:1], 1e-12)
    x = jnp.concatenate([lo_ref[...], hi_ref[...]], axis=1)
    x1 = jax.nn.sigmoid(x * dinv + b_ref[...])
    bm = jnp.max(x1, axis=0, keepdims=True)
    m_ref[0:1, :] = jnp.maximum(m_ref[0:1, :], bm)

    @pl.when(i == N_NODES // _BR - 1)
    def _():
        o_ref[...] = jnp.dot(m_ref[0:1, :], wl_ref[...],
                             preferred_element_type=jnp.float32,
                             precision=lax.Precision.HIGHEST) + bl_ref[...]


_k_fin = pl.pallas_call(
    _fin_body,
    grid=(N_NODES // _BR,),
    in_specs=[
        pl.BlockSpec((_BR, 64), lambda i: (i, 0)),
        pl.BlockSpec((_BR, 64), lambda i: (i, 0)),
        pl.BlockSpec((_BR, 16), lambda i: (i, 0)),
        pl.BlockSpec((1, HID), lambda i: (0, 0)),
        pl.BlockSpec((HID, 1), lambda i: (0, 0)),
        pl.BlockSpec((1, 1), lambda i: (0, 0)),
    ],
    out_specs=pl.BlockSpec((1, 1), lambda i: (0, 0)),
    out_shape=_f32((1, 1)),
    scratch_shapes=[pltpu.VMEM((8, HID), jnp.float32)],
)


def kernel(x_0, node_idx, edge_idx, W01_0, b1_0, W10_0, b0_0,
           W01_1, b1_1, W10_1, b0_1, W_lin, b_lin):
    zeros16 = jnp.zeros((OUT_SLICE, 16), jnp.float32)
    zeros32 = jnp.zeros((OUT_SLICE, QW), jnp.float32)
    one = jnp.sum(b_lin) * 0.0 + 1.0
    ones_n = jnp.full((N_NODES, 16), 1.0, jnp.float32) * one
    ones_e = jnp.full((N_EDGES, 16), 1.0, jnp.float32) * one

    ecnt, ncnt = _k_scalar_sums(ones_n, ones_e, node_idx, edge_idx, zeros16)
    te, tn = _k_norm(ecnt, ncnt)
    d1s, d0s = _k_scalar_sums(_chain(tn, ecnt), te, node_idx, edge_idx,
                              zeros16)

    zeros64 = jnp.zeros((OUT_SLICE, 64), jnp.float32)

    def seg_pass_e(q, token):
        a0, a1 = _k_pass(_chain(q[0], token), q[1], node_idx, edge_idx,
                         zeros32)
        a2, a3 = _k_pass(_chain(q[2], a0), q[3], node_idx, edge_idx, zeros32)
        return a0, a1, a2, a3

    xb = _k_in(x_0, W01_0, tn)
    aa = seg_pass_e(xb, d0s)
    xe = _k_mid_e(*aa, d1s, b1_0.reshape(1, -1), W10_0, te)
    ab = _k_pass_n(_chain(xe[0], aa[2]), xe[1], edge_idx, node_idx, zeros64)
    xb = _k_mid_n(*ab, d0s, b0_0.reshape(1, -1), W01_1, tn)
    aa = seg_pass_e(xb, ab[0])
    xe = _k_mid_e(*aa, d1s, b1_1.reshape(1, -1), W10_1, te)
    ab = _k_pass_n(_chain(xe[0], aa[2]), xe[1], edge_idx, node_idx, zeros64)

    out = _k_fin(*ab, d0s, b0_1.reshape(1, -1), W_lin, b_lin.reshape(1, 1))
    return out.reshape(1)


# speedup vs baseline: 1.2497x; 1.0400x over previous
import functools

import jax
import jax.numpy as jnp
from jax import lax
from jax.experimental import pallas as pl
from jax.experimental.pallas import tpu as pltpu
from jax.experimental.pallas import tpu_sc as plsc

N_NODES = 10000
N_EDGES = 20000
NNZ = 320000
HID = 128
QW = 32

NS = 16
PER_W = NNZ // NS
CHUNK = 1000
ITERS = PER_W // CHUNK
CHUNK_P = 1000
ITERS_P = PER_W // CHUNK_P
OUT_SLICE = 2000

_mesh = plsc.VectorSubcoreMesh(core_axis_name="c", subcore_axis_name="s")

_SC_PARAMS = pltpu.CompilerParams(use_tc_tiling_on_sc=False)


def _f32(shape):
    return jax.ShapeDtypeStruct(shape, jnp.float32)


def _chain(x, token):
    return lax.optimization_barrier((x, token))[0]


def _zero_acc(acc, zeros_hbm, s):
    @pl.when(s < N_EDGES // OUT_SLICE)
    def _():
        pltpu.sync_copy(zeros_hbm, acc.at[pl.ds(s * OUT_SLICE, OUT_SLICE)])


def _copy_out(acc, out, s, cond):
    @pl.when(cond)
    def _():
        sl = pl.ds(s * OUT_SLICE, OUT_SLICE)
        pltpu.sync_copy(acc.at[sl], out.at[sl])


def _load_idx2d(idx_hbm, buf, s, chunk, iters):
    base = s * PER_W

    @pl.loop(0, iters)
    def _(i):
        pltpu.sync_copy(idx_hbm.at[pl.ds(base + i * chunk, chunk)], buf.at[i])


def _run_phase(tbl, acc, src2d, dst2d, rows0, rows1, sem0, sem1, iters):
    @pl.loop(0, iters)
    def _(i):
        pltpu.sync_copy(tbl.at[src2d.at[i]], rows0)
        pltpu.sync_copy(rows0, acc.at[dst2d.at[i]], add=True)


def _sc_scratch(width, chunk, iters, acc_rows=N_EDGES):
    return [
        pltpu.VMEM((iters, chunk), jnp.int32),
        pltpu.VMEM((iters, chunk), jnp.int32),
        pltpu.VMEM((chunk, width), jnp.float32),
        pltpu.VMEM((chunk, width), jnp.float32),
        pltpu.VMEM_SHARED((acc_rows, width), jnp.float32),
        pltpu.SemaphoreType.DMA,
        pltpu.SemaphoreType.DMA,
    ]


@functools.partial(
    pl.kernel,
    out_type=(_f32((N_EDGES, 16)), _f32((N_NODES, 16))),
    mesh=_mesh,
    compiler_params=_SC_PARAMS,
    scratch_types=_sc_scratch(16, CHUNK, ITERS),
)
def _k_scalar_sums(tbl_n, tbl_e, nidx, eidx, zeros_hbm, oute, outn,
                   buf_n, buf_e, rows0, rows1, acc, sem0, sem1):
    c = lax.axis_index("c")
    s = lax.axis_index("s")

    _zero_acc(acc, zeros_hbm, s)
    _load_idx2d(nidx, buf_n, s, CHUNK, ITERS)
    _load_idx2d(eidx, buf_e, s, CHUNK, ITERS)
    plsc.subcore_barrier()

    pl.when(c == 0)(lambda: _run_phase(
        tbl_n, acc, buf_n, buf_e, rows0, rows1, sem0, sem1, ITERS))
    pl.when(c == 1)(lambda: _run_phase(
        tbl_e, acc, buf_e, buf_n, rows0, rows1, sem0, sem1, ITERS))
    plsc.subcore_barrier()

    _copy_out(acc, oute, s, jnp.logical_and(c == 0, s < N_EDGES // OUT_SLICE))
    _copy_out(acc, outn, s, jnp.logical_and(c == 1, s < N_NODES // OUT_SLICE))


@functools.partial(
    pl.kernel,
    out_type=tuple(_f32((N_EDGES, QW)) for _ in range(4)),
    mesh=_mesh,
    compiler_params=_SC_PARAMS,
    scratch_types=_sc_scratch(QW, CHUNK_P, ITERS_P),
)
def _k_pass(q0, q1, q2, q3, sidx, didx, zeros_hbm, o0, o1, o2, o3,
            buf_s, buf_d, rows0, rows1, acc, sem0, sem1):
    c = lax.axis_index("c")
    s = lax.axis_index("s")
    is0 = jnp.logical_and(c == 0, s < N_EDGES // OUT_SLICE)
    is1 = jnp.logical_and(c == 1, s < N_EDGES // OUT_SLICE)

    _zero_acc(acc, zeros_hbm, s)
    _load_idx2d(sidx, buf_s, s, CHUNK_P, ITERS_P)
    _load_idx2d(didx, buf_d, s, CHUNK_P, ITERS_P)
    plsc.subcore_barrier()

    pl.when(c == 0)(lambda: _run_phase(
        q0, acc, buf_s, buf_d, rows0, rows1, sem0, sem1, ITERS_P))
    pl.when(c == 1)(lambda: _run_phase(
        q1, acc, buf_s, buf_d, rows0, rows1, sem0, sem1, ITERS_P))
    plsc.subcore_barrier()
    _copy_out(acc, o0, s, is0)
    _copy_out(acc, o1, s, is1)
    plsc.subcore_barrier()
    _zero_acc(acc, zeros_hbm, s)
    plsc.subcore_barrier()
    pl.when(c == 0)(lambda: _run_phase(
        q2, acc, buf_s, buf_d, rows0, rows1, sem0, sem1, ITERS_P))
    pl.when(c == 1)(lambda: _run_phase(
        q3, acc, buf_s, buf_d, rows0, rows1, sem0, sem1, ITERS_P))
    plsc.subcore_barrier()
    _copy_out(acc, o2, s, is0)
    _copy_out(acc, o3, s, is1)


@functools.partial(
    pl.kernel,
    out_type=(_f32((N_NODES, 64)), _f32((N_NODES, 64))),
    mesh=_mesh,
    compiler_params=_SC_PARAMS,
    scratch_types=_sc_scratch(64, 400, PER_W // 400, N_NODES),
)
def _k_pass_n(tbl_lo, tbl_hi, sidx, didx, zeros_hbm, out_lo, out_hi,
              buf_s, buf_d, rows0, rows1, acc, sem0, sem1):
    c = lax.axis_index("c")
    s = lax.axis_index("s")
    it = PER_W // 400

    @pl.when(s < N_NODES // OUT_SLICE)
    def _():
        pltpu.sync_copy(zeros_hbm, acc.at[pl.ds(s * OUT_SLICE, OUT_SLICE)])
    _load_idx2d(sidx, buf_s, s, 400, it)
    _load_idx2d(didx, buf_d, s, 400, it)
    plsc.subcore_barrier()

    pl.when(c == 0)(lambda: _run_phase(
        tbl_lo, acc, buf_s, buf_d, rows0, rows1, sem0, sem1, it))
    pl.when(c == 1)(lambda: _run_phase(
        tbl_hi, acc, buf_s, buf_d, rows0, rows1, sem0, sem1, it))
    plsc.subcore_barrier()

    _copy_out(acc, out_lo, s, jnp.logical_and(c == 0, s < N_NODES // OUT_SLICE))
    _copy_out(acc, out_hi, s, jnp.logical_and(c == 1, s < N_NODES // OUT_SLICE))


def _norm_body(ec_ref, nc_ref, te_ref, tn_ref):
    r = lax.rsqrt(jnp.maximum(ec_ref[...], 1.0))
    te_ref[...] = r * r * r
    tn_ref[...] = lax.rsqrt(jnp.maximum(nc_ref[...], 1.0))


_k_norm = pl.pallas_call(
    _norm_body,
    out_shape=(_f32((N_EDGES, 16)), _f32((N_NODES, 16))),
)


_BR = 2000


def _write_quarters(y, out_refs):
    for j, o_ref in enumerate(out_refs):
        o_ref[...] = y[:, j * QW:(j + 1) * QW]


def _in_body(x_ref, w_ref, t_ref, *out_refs):
    y = jnp.dot(x_ref[...], w_ref[...], preferred_element_type=jnp.float32,
                precision=lax.Precision.HIGHEST)
    _write_quarters(y * t_ref[:, 0:1], out_refs)


_h_specs = tuple(pl.BlockSpec((_BR, QW), lambda i: (i, 0)) for _ in range(4))
_h_shapes = tuple(_f32((N_EDGES, QW)) for _ in range(4))

_k_in = pl.pallas_call(
    _in_body,
    grid=(N_NODES // _BR,),
    in_specs=[
        pl.BlockSpec((_BR, HID), lambda i: (i, 0)),
        pl.BlockSpec((HID, HID), lambda i: (0, 0)),
        pl.BlockSpec((_BR, 16), lambda i: (i, 0)),
    ],
    out_specs=_h_specs,
    out_shape=_h_shapes,
)


def _make_mid(n_rows, in_widths, out_widths, out_rows):
    def body(*refs):
        a = refs[:len(in_widths)]
        d_ref, b_ref, w_ref, t_ref = refs[len(in_widths):len(in_widths) + 4]
        out_refs = refs[len(in_widths) + 4:]
        dinv = 1.0 / jnp.maximum(d_ref[:, 0:1], 1e-12)
        x = jnp.concatenate([r[...] for r in a], axis=1)
        x1 = jax.nn.sigmoid(x * dinv + b_ref[...])
        y = jnp.dot(x1, w_ref[...], preferred_element_type=jnp.float32,
                    precision=lax.Precision.HIGHEST)
        y = y * t_ref[:, 0:1]
        off = 0
        for w, o_ref in zip(out_widths, out_refs):
            o_ref[...] = y[:, off:off + w]
            off += w

    return pl.pallas_call(
        body,
        grid=(n_rows // _BR,),
        in_specs=[
            *(pl.BlockSpec((_BR, w), lambda i: (i, 0)) for w in in_widths),
            pl.BlockSpec((_BR, 16), lambda i: (i, 0)),
            pl.BlockSpec((1, HID), lambda i: (0, 0)),
            pl.BlockSpec((HID, HID), lambda i: (0, 0)),
            pl.BlockSpec((_BR, 16), lambda i: (i, 0)),
        ],
        out_specs=tuple(pl.BlockSpec((_BR, w), lambda i: (i, 0))
                        for w in out_widths),
        out_shape=tuple(_f32((out_rows, w)) for w in out_widths),
    )


_k_mid_e = _make_mid(N_EDGES, (QW,) * 4, (64, 64), N_EDGES)
_k_mid_n = _make_mid(N_NODES, (64, 64), (QW,) * 4, N_EDGES)


def _fin_body(lo_ref, hi_ref, d_ref, b_ref, wl_ref, bl_ref, o_ref, m_ref):
    i = pl.program_id(0)

    @pl.when(i == 0)
    def _():
        m_ref[...] = jnp.full((8, HID), -jnp.inf, jnp.float32)

    dinv = 1.0 / jnp.maximum(d_ref[:, 0:1], 1e-12)
    x = jnp.concatenate([lo_ref[...], hi_ref[...]], axis=1)
    x1 = jax.nn.sigmoid(x * dinv + b_ref[...])
    bm = jnp.max(x1, axis=0, keepdims=True)
    m_ref[0:1, :] = jnp.maximum(m_ref[0:1, :], bm)

    @pl.when(i == N_NODES // _BR - 1)
    def _():
        o_ref[...] = jnp.dot(m_ref[0:1, :], wl_ref[...],
                             preferred_element_type=jnp.float32,
                             precision=lax.Precision.HIGHEST) + bl_ref[...]


_k_fin = pl.pallas_call(
    _fin_body,
    grid=(N_NODES // _BR,),
    in_specs=[
        pl.BlockSpec((_BR, 64), lambda i: (i, 0)),
        pl.BlockSpec((_BR, 64), lambda i: (i, 0)),
        pl.BlockSpec((_BR, 16), lambda i: (i, 0)),
        pl.BlockSpec((1, HID), lambda i: (0, 0)),
        pl.BlockSpec((HID, 1), lambda i: (0, 0)),
        pl.BlockSpec((1, 1), lambda i: (0, 0)),
    ],
    out_specs=pl.BlockSpec((1, 1), lambda i: (0, 0)),
    out_shape=_f32((1, 1)),
    scratch_shapes=[pltpu.VMEM((8, HID), jnp.float32)],
)


def kernel(x_0, node_idx, edge_idx, W01_0, b1_0, W10_0, b0_0,
           W01_1, b1_1, W10_1, b0_1, W_lin, b_lin):
    zeros16 = jnp.zeros((OUT_SLICE, 16), jnp.float32)
    zeros32 = jnp.zeros((OUT_SLICE, QW), jnp.float32)
    one = jnp.sum(b_lin) * 0.0 + 1.0
    ones_n = jnp.full((N_NODES, 16), 1.0, jnp.float32) * one
    ones_e = jnp.full((N_EDGES, 16), 1.0, jnp.float32) * one

    ecnt, ncnt = _k_scalar_sums(ones_n, ones_e, node_idx, edge_idx, zeros16)
    te, tn = _k_norm(ecnt, ncnt)
    d1s, d0s = _k_scalar_sums(_chain(tn, ecnt), te, node_idx, edge_idx,
                              zeros16)

    zeros64 = jnp.zeros((OUT_SLICE, 64), jnp.float32)

    def seg_pass_e(q, token):
        return _k_pass(_chain(q[0], token), q[1], q[2], q[3],
                       node_idx, edge_idx, zeros32)

    xb = _k_in(x_0, W01_0, tn)
    aa = seg_pass_e(xb, d0s)
    xe = _k_mid_e(*aa, d1s, b1_0.reshape(1, -1), W10_0, te)
    ab = _k_pass_n(_chain(xe[0], aa[2]), xe[1], edge_idx, node_idx, zeros64)
    xb = _k_mid_n(*ab, d0s, b0_0.reshape(1, -1), W01_1, tn)
    aa = seg_pass_e(xb, ab[0])
    xe = _k_mid_e(*aa, d1s, b1_1.reshape(1, -1), W10_1, te)
    ab = _k_pass_n(_chain(xe[0], aa[2]), xe[1], edge_idx, node_idx, zeros64)

    out = _k_fin(*ab, d0s, b0_1.reshape(1, -1), W_lin, b_lin.reshape(1, 1))
    return out.reshape(1)
